# baseline (speedup 1.0000x reference)
import functools

import numpy as np
import jax
import jax.numpy as jnp
from jax import lax
from jax.experimental import pallas as pl
from jax.experimental.pallas import tpu as pltpu
from jax.experimental.pallas import tpu_sc as plsc

D = 128
NW = 32
R = 160
BLK = 320
CHUNK = 128
_AVG_LOG = float(np.mean(np.log(np.array([16.0, 32.0, 32.0, 48.0]) + 1.0)))


def _pre_body(x_ref, w0_ref, wp1_ref, wp2_ref, m_ref, a_ref, b_ref):
    m = jnp.dot(x_ref[...], w0_ref[...], preferred_element_type=jnp.float32)
    m_ref[...] = m
    a_ref[...] = jnp.dot(m, wp1_ref[...], preferred_element_type=jnp.float32)
    b_ref[...] = jnp.dot(m, wp2_ref[...], preferred_element_type=jnp.float32)


def _pre(x, w0, wp1, wp2):
    n = x.shape[0]
    bn = 1000 if n % 1000 == 0 else n
    grid = n // bn
    out = jax.ShapeDtypeStruct((n, D), jnp.float32)
    row_spec = pl.BlockSpec((bn, D), lambda i: (i, 0))
    full = pl.BlockSpec((D, D), lambda i: (0, 0))
    return pl.pallas_call(
        _pre_body,
        grid=(grid,),
        in_specs=[row_spec, full, full, full],
        out_specs=[row_spec, row_spec, row_spec],
        out_shape=[out, out, out],
    )(x, w0, wp1, wp2)


def _consts_body(we_ref, be_ref, bp_ref, wp3_ref, o_ref):
    w2 = jnp.dot(we_ref[...], wp3_ref[...], preferred_element_type=jnp.float32)
    cb = jnp.dot(be_ref[...], wp3_ref[...], preferred_element_type=jnp.float32)
    o_ref[...] = jnp.concatenate(
        [w2[0:3], cb[0:1], bp_ref[0:1], jnp.zeros((3, D), jnp.float32)], axis=0)


def _consts(w_edge, b_edge, b_pre, wp3):
    we8 = jnp.zeros((8, D), jnp.float32).at[0:3].set(w_edge)
    be8 = jnp.zeros((8, D), jnp.float32).at[0].set(b_edge)
    bp8 = jnp.zeros((8, D), jnp.float32).at[0].set(b_pre)
    return pl.pallas_call(
        _consts_body,
        out_shape=jax.ShapeDtypeStruct((8, D), jnp.float32),
    )(we8, be8, bp8, wp3)


def _tmat_body(arr_ref, c_ref, t_ref):
    blk = arr_ref[...]
    eww = blk[4:5, :]
    s4 = blk[0:4, :] * eww
    s8 = jnp.concatenate([s4, jnp.zeros_like(s4)], axis=0)
    mmat = jnp.concatenate([c_ref[0:4], jnp.zeros((4, D), jnp.float32)], axis=0)
    t = lax.dot_general(s8, mmat, (((0,), (0,)), ((), ())),
                        preferred_element_type=jnp.float32)
    t_ref[...] = t + c_ref[4:5, :]


def _tmat(arr8, consts):
    e = arr8.shape[1]
    be = 16000
    grid = e // be
    return pl.pallas_call(
        _tmat_body,
        grid=(grid,),
        in_specs=[pl.BlockSpec((8, be), lambda i: (0, i)),
                  pl.BlockSpec((8, D), lambda i: (0, 0))],
        out_specs=pl.BlockSpec((be, D), lambda i: (i, 0)),
        out_shape=jax.ShapeDtypeStruct((e, D), jnp.float32),
    )(arr8, consts)


NGRP = BLK // 16


def _sc_body(dst_h, src_h, ew_h, A_h, B_h, T_h,
             osum, osq, omin, omax, ocnt,
             accs, accq, accn, accx, cntv, dstblk2, gaccb,
             ev_v2, dv_v2, svg2, ewg2, arows2, brows2, trows2,
             pend_dv, pend_e, sew, sem, semd):
    E = dst_h.shape[0]
    N = A_h.shape[0]
    nblk = E // BLK
    cid = lax.axis_index("c")
    sid = lax.axis_index("s")
    wid = sid * 2 + cid

    zero16 = jnp.zeros((16,), jnp.float32)
    inf16 = jnp.full((16,), jnp.inf, jnp.float32)
    e0vec = jnp.where(lax.iota(jnp.int32, 16) == 0,
                      jnp.float32(1.0), jnp.float32(0.0))
    iota16 = lax.iota(jnp.int32, 16)

    def _issue(head, par):
        dvv = jnp.zeros((16,), jnp.int32)
        evv = jnp.zeros((16,), jnp.int32)
        for t in range(16):
            dvv = jnp.where(iota16 == t, pend_dv[head + t], dvv)
            evv = jnp.where(iota16 == t, pend_e[head + t], evv)
        dv_v2[par, :] = jnp.minimum(dvv, N - 1)
        ev_v2[par, :] = evv
        pb = pl.multiple_of(par * 16, 16)
        cp = pltpu.async_copy(src_h.at[ev_v2.at[par]], svg2.at[par],
                              sem.at[par])
        cp.wait()
        pltpu.async_copy(ew_h.at[ev_v2.at[par]], ewg2.at[par], sem.at[par])
        pltpu.async_copy(A_h.at[dv_v2.at[par]],
                         arows2.at[pl.ds(pb, 16)], sem.at[par])
        pltpu.async_copy(T_h.at[ev_v2.at[par]],
                         trows2.at[pl.ds(pb, 16)], sem.at[par])
        pltpu.async_copy(B_h.at[svg2.at[par]],
                         brows2.at[pl.ds(pb, 16)], sem.at[par])

    def _work(i, args):
        lo, nfull16 = args
        j = i & 15

        @pl.when(j == 0)
        def _():
            head = pl.multiple_of(i & ~15, 16)
            par = (i >> 4) & 1
            pb = pl.multiple_of(par * 16, 16)
            pltpu.make_async_copy(ew_h.at[ev_v2.at[par]], ewg2.at[par],
                                  sem.at[par]).wait()
            pltpu.make_async_copy(A_h.at[dv_v2.at[par]],
                                  arows2.at[pl.ds(pb, 16)], sem.at[par]).wait()
            pltpu.make_async_copy(T_h.at[ev_v2.at[par]],
                                  trows2.at[pl.ds(pb, 16)], sem.at[par]).wait()
            pltpu.make_async_copy(B_h.at[svg2.at[par]],
                                  brows2.at[pl.ds(pb, 16)], sem.at[par]).wait()
            ewr = ewg2[par, :]
            for t in range(16):
                sew[pb + t] = ewr[t]

            @pl.when(head + 16 < nfull16)
            def _():
                _issue(head + 16, par ^ 1)

        par = (i >> 4) & 1
        pb2 = pl.multiple_of(par * 16, 16)
        dl = pend_dv[i] - lo
        ew = sew[pb2 + j]
        plsc.addupdate(cntv.at[dl], e0vec)
        for q in range(D // 16):
            sl = pl.ds(q * 16, 16)
            v = ew * (arows2[pb2 + j, sl] + brows2[pb2 + j, sl]) \
                + trows2[pb2 + j, sl]
            plsc.addupdate(accs.at[dl, sl], v)
            plsc.addupdate(accq.at[dl, sl], v * v)
            accn[dl, sl] = jnp.minimum(accn[dl, sl], v)
            accx[dl, sl] = jnp.maximum(accx[dl, sl], v)

    def _popcount(x):
        x = x - ((x >> 1) & 0x55555555)
        x = (x & 0x33333333) + ((x >> 2) & 0x33333333)
        x = (x + (x >> 4)) & 0x0F0F0F0F
        return (x * 0x01010101) >> 24

    for p in range(2):
        lo = (p * NW + wid) * R
        hi = lo + R

        def _init_acc(r, _):
            for q in range(D // 16):
                sl = pl.ds(q * 16, 16)
                accs[r, sl] = zero16
                accq[r, sl] = zero16
                accn[r, sl] = inf16
                accx[r, sl] = -inf16
            cntv[r, :] = zero16
            return 0
        lax.fori_loop(0, R + 1, _init_acc, 0)

        pltpu.async_copy(dst_h.at[pl.ds(0, BLK)],
                         dstblk2.at[pl.ds(0, BLK)], semd.at[0])

        def _block(b, n):
            boff = pl.multiple_of(b * BLK, BLK)
            bpar = b & 1
            bb = pl.multiple_of(bpar * BLK, BLK)
            pltpu.make_async_copy(dst_h.at[pl.ds(boff, BLK)],
                                  dstblk2.at[pl.ds(bb, BLK)],
                                  semd.at[bpar]).wait()

            @pl.when(b + 1 < nblk)
            def _():
                boff2 = pl.multiple_of((b + 1) * BLK, BLK)
                bb2 = pl.multiple_of((1 - bpar) * BLK, BLK)
                pltpu.async_copy(dst_h.at[pl.ds(boff2, BLK)],
                                 dstblk2.at[pl.ds(bb2, BLK)],
                                 semd.at[1 - bpar])
            gacc = jnp.zeros((16,), jnp.int32)
            for k in range(NGRP):
                dv = dstblk2[pl.ds(bb + k * 16, 16)]
                msk = jnp.where((dv >= lo) & (dv < hi),
                                jnp.int32(1 << k), jnp.int32(0))
                gacc = gacc | msk
            gaccb[...] = gacc
            gv = gaccb[...]
            for l in range(16):
                gl0 = gv[l]

                def _bit(_, carry):
                    gl, nn = carry
                    bit = gl & (-gl)
                    fb = lax.bitcast_convert_type(
                        bit.astype(jnp.float32), jnp.int32)
                    k = (fb >> 23) - 127
                    ko = pl.multiple_of(k * 16, 16)
                    dvk = dstblk2[pl.ds(pl.multiple_of(bb + ko, 16), 16)]
                    pend_dv[nn] = dvk[l]
                    pend_e[nn] = boff + k * 16 + l
                    return (gl ^ bit, nn + 1)

                (_, n) = lax.fori_loop(0, _popcount(gl0), _bit, (gl0, n))
            nfull16 = n & ~15

            @pl.when(nfull16 > 0)
            def _():
                _issue(0, 0)

            def _w(i, _):
                _work(i, (lo, nfull16))
                return 0
            lax.fori_loop(0, nfull16, _w, 0)
            nrem = n - nfull16

            def _mv(i, _):
                pend_dv[i] = pend_dv[nfull16 + i]
                pend_e[i] = pend_e[nfull16 + i]
                return 0
            lax.fori_loop(0, nrem, _mv, 0)
            return nrem

        n = lax.fori_loop(0, nblk, _block, jnp.int32(0))

        @pl.when(n > 0)
        def _():
            def _pad(i, _):
                @pl.when(i >= n)
                def _():
                    pend_dv[i] = lo + R
                    pend_e[i] = 0
                return 0
            lax.fori_loop(0, 16, _pad, 0)
            _issue(0, 0)

            def _w2(i, _):
                _work(i, (lo, 16))
                return 0
            lax.fori_loop(0, 16, _w2, 0)

        obase = (p * NW + wid) * R
        pltpu.sync_copy(accs.at[pl.ds(0, R)], osum.at[pl.ds(obase, R)])
        pltpu.sync_copy(accq.at[pl.ds(0, R)], osq.at[pl.ds(obase, R)])
        pltpu.sync_copy(accn.at[pl.ds(0, R)], omin.at[pl.ds(obase, R)])
        pltpu.sync_copy(accx.at[pl.ds(0, R)], omax.at[pl.ds(obase, R)])
        pltpu.sync_copy(cntv.at[pl.ds(0, R)], ocnt.at[pl.ds(obase, R)])


def _sc_agg(dst, src, ew, A, B, T):
    ntot = 2 * NW * R
    outs = (
        jax.ShapeDtypeStruct((ntot, D), jnp.float32),
        jax.ShapeDtypeStruct((ntot, D), jnp.float32),
        jax.ShapeDtypeStruct((ntot, D), jnp.float32),
        jax.ShapeDtypeStruct((ntot, D), jnp.float32),
        jax.ShapeDtypeStruct((ntot, 16), jnp.float32),
    )
    mesh = plsc.VectorSubcoreMesh(core_axis_name="c", subcore_axis_name="s",
                                  num_cores=2, num_subcores=16)
    scratch = [
        pltpu.VMEM((R + 8, D), jnp.float32),
        pltpu.VMEM((R + 8, D), jnp.float32),
        pltpu.VMEM((R + 8, D), jnp.float32),
        pltpu.VMEM((R + 8, D), jnp.float32),
        pltpu.VMEM((R + 8, 16), jnp.float32),
        pltpu.VMEM((2 * BLK,), jnp.int32),
        pltpu.VMEM((16,), jnp.int32),
        pltpu.VMEM((2, 16), jnp.int32),
        pltpu.VMEM((2, 16), jnp.int32),
        pltpu.VMEM((2, 16), jnp.int32),
        pltpu.VMEM((2, 16), jnp.float32),
        pltpu.VMEM((32, D), jnp.float32),
        pltpu.VMEM((32, D), jnp.float32),
        pltpu.VMEM((32, D), jnp.float32),
        pltpu.SMEM((BLK + 16,), jnp.int32),
        pltpu.SMEM((BLK + 16,), jnp.int32),
        pltpu.SMEM((32,), jnp.float32),
        pltpu.SemaphoreType.DMA((2,)),
        pltpu.SemaphoreType.DMA((2,)),
    ]
    fn = pl.kernel(_sc_body, out_type=outs, mesh=mesh, scratch_types=scratch)
    return fn(dst, src, ew, A, B, T)


def _post_body(m_ref, x_ref, s_ref, q_ref, mn_ref, mx_ref, c_ref,
               pm_ref, pagg_ref, pamp_ref, patt_ref, bias_ref,
               wih_ref, whh_ref, o_ref):
    cnt = c_ref[:, 0:1]
    deg = jnp.maximum(cnt, 1.0)
    inv = 1.0 / deg
    mean = s_ref[...] * inv
    msq = q_ref[...] * inv
    var = jnp.maximum(msq - mean * mean, 0.0)
    std = jnp.sqrt(var + 1e-5)
    mn = mn_ref[...]
    mn = jnp.where(jnp.isfinite(mn), mn, 0.0)
    mx = mx_ref[...]
    mx = jnp.where(jnp.isfinite(mx), mx, 0.0)
    agg = jnp.concatenate([mean, mn, mx, std], axis=1)
    ld = jnp.log(deg + 1.0)
    out = jnp.dot(m_ref[...], pm_ref[...], preferred_element_type=jnp.float32)
    out = out + jnp.dot(agg, pagg_ref[...], preferred_element_type=jnp.float32)
    out = out + jnp.dot(agg * (ld / _AVG_LOG), pamp_ref[...],
                        preferred_element_type=jnp.float32)
    out = out + jnp.dot(agg * (_AVG_LOG / ld), patt_ref[...],
                        preferred_element_type=jnp.float32)
    out = out + bias_ref[0:1, :]
    gi = jnp.dot(out, wih_ref[...], preferred_element_type=jnp.float32)
    gh = jnp.dot(x_ref[...], whh_ref[...], preferred_element_type=jnp.float32)
    i_r = gi[:, 0:D] + bias_ref[1:2, :]
    i_z = gi[:, D:2 * D] + bias_ref[2:3, :]
    i_n = gi[:, 2 * D:] + bias_ref[3:4, :]
    h_r = gh[:, 0:D] + bias_ref[4:5, :]
    h_z = gh[:, D:2 * D] + bias_ref[5:6, :]
    h_n = gh[:, 2 * D:] + bias_ref[6:7, :]
    r = jax.nn.sigmoid(i_r + h_r)
    z = jax.nn.sigmoid(i_z + h_z)
    nn_ = jnp.tanh(i_n + r * h_n)
    o_ref[...] = (1.0 - z) * nn_ + z * x_ref[...]


def _post(m, x, s, q, mn, mx, cnt128, w_post, b_post, w_ih, w_hh, b_ih, b_hh):
    n = x.shape[0]
    bn = 1000 if n % 1000 == 0 else n
    grid = n // bn
    pm = w_post[0:D]
    pagg = w_post[D:5 * D]
    pamp = w_post[5 * D:9 * D]
    patt = w_post[9 * D:13 * D]
    bias = jnp.zeros((8, D), jnp.float32)
    bias = bias.at[0].set(b_post)
    bias = bias.at[1:4].set(b_ih.reshape(3, D))
    bias = bias.at[4:7].set(b_hh.reshape(3, D))
    row = pl.BlockSpec((bn, D), lambda i: (i, 0))
    f = lambda shape: pl.BlockSpec(shape, lambda i: (0, 0))
    return pl.pallas_call(
        _post_body,
        grid=(grid,),
        in_specs=[row, row, row, row, row, row, row,
                  f((D, D)), f((4 * D, D)), f((4 * D, D)), f((4 * D, D)),
                  f((8, D)), f((D, 3 * D)), f((D, 3 * D))],
        out_specs=row,
        out_shape=jax.ShapeDtypeStruct((n, D), jnp.float32),
    )(m, x, s, q, mn, mx, cnt128, pm, pagg, pamp, patt, bias,
      w_ih.T, w_hh.T)


@jax.jit
def kernel(x, edge_index, edge_attr, weight, W_edge, b_edge, W_pre, b_pre,
           W_post, b_post, w_ih, w_hh, b_ih, b_hh):
    n = x.shape[0]
    src = edge_index[0]
    dst = edge_index[1]
    wp1 = W_pre[0:D]
    wp2 = W_pre[D:2 * D]
    wp3 = W_pre[2 * D:]
    m, A, B = _pre(x, weight[0], wp1, wp2)
    consts = _consts(W_edge, b_edge, b_pre, wp3)
    e = edge_attr.shape[0]
    ew = edge_attr[:, 3]
    arr8 = jnp.concatenate([
        edge_attr[:, 0:3].T, jnp.ones((1, e), jnp.float32),
        ew[None, :], jnp.zeros((3, e), jnp.float32)], axis=0)
    T = _tmat(arr8, consts)
    s, q, mn, mx, cnt = _sc_agg(dst, src, ew, A, B, T)
    cnt128 = jnp.pad(cnt[:n], ((0, 0), (0, D - 16)))
    return _post(m, x, s[:n], q[:n], mn[:n], mx[:n], cnt128,
                 W_post, b_post, w_ih, w_hh, b_ih, b_hh)


# speedup vs baseline: 1.0812x; 1.0812x over previous
import functools

import numpy as np
import jax
import jax.numpy as jnp
from jax import lax
from jax.experimental import pallas as pl
from jax.experimental.pallas import tpu as pltpu
from jax.experimental.pallas import tpu_sc as plsc

D = 128
NW = 32
R = 160
BLK = 320
CHUNK = 128
_AVG_LOG = float(np.mean(np.log(np.array([16.0, 32.0, 32.0, 48.0]) + 1.0)))


def _pre_body(x_ref, w0_ref, wp1_ref, wp2_ref, m_ref, a_ref, b_ref):
    m = jnp.dot(x_ref[...], w0_ref[...], preferred_element_type=jnp.float32)
    m_ref[...] = m
    a_ref[...] = jnp.dot(m, wp1_ref[...], preferred_element_type=jnp.float32)
    b_ref[...] = jnp.dot(m, wp2_ref[...], preferred_element_type=jnp.float32)


def _pre(x, w0, wp1, wp2):
    n = x.shape[0]
    bn = 1000 if n % 1000 == 0 else n
    grid = n // bn
    out = jax.ShapeDtypeStruct((n, D), jnp.float32)
    row_spec = pl.BlockSpec((bn, D), lambda i: (i, 0))
    full = pl.BlockSpec((D, D), lambda i: (0, 0))
    return pl.pallas_call(
        _pre_body,
        grid=(grid,),
        in_specs=[row_spec, full, full, full],
        out_specs=[row_spec, row_spec, row_spec],
        out_shape=[out, out, out],
    )(x, w0, wp1, wp2)


def _consts_body(we_ref, be_ref, bp_ref, wp3_ref, o_ref):
    w2 = jnp.dot(we_ref[...], wp3_ref[...], preferred_element_type=jnp.float32)
    cb = jnp.dot(be_ref[...], wp3_ref[...], preferred_element_type=jnp.float32)
    o_ref[...] = jnp.concatenate(
        [w2[0:3], cb[0:1], bp_ref[0:1], jnp.zeros((3, D), jnp.float32)], axis=0)


def _consts(w_edge, b_edge, b_pre, wp3):
    we8 = jnp.zeros((8, D), jnp.float32).at[0:3].set(w_edge)
    be8 = jnp.zeros((8, D), jnp.float32).at[0].set(b_edge)
    bp8 = jnp.zeros((8, D), jnp.float32).at[0].set(b_pre)
    return pl.pallas_call(
        _consts_body,
        out_shape=jax.ShapeDtypeStruct((8, D), jnp.float32),
    )(we8, be8, bp8, wp3)


def _tmat_body(arr_ref, c_ref, t_ref):
    blk = arr_ref[...]
    eww = blk[4:5, :]
    s4 = blk[0:4, :] * eww
    s8 = jnp.concatenate([s4, jnp.zeros_like(s4)], axis=0)
    mmat = jnp.concatenate([c_ref[0:4], jnp.zeros((4, D), jnp.float32)], axis=0)
    t = lax.dot_general(s8, mmat, (((0,), (0,)), ((), ())),
                        preferred_element_type=jnp.float32)
    t_ref[...] = t + c_ref[4:5, :]


def _tmat(arr8, consts):
    e = arr8.shape[1]
    be = 16000
    grid = e // be
    return pl.pallas_call(
        _tmat_body,
        grid=(grid,),
        in_specs=[pl.BlockSpec((8, be), lambda i: (0, i)),
                  pl.BlockSpec((8, D), lambda i: (0, 0))],
        out_specs=pl.BlockSpec((be, D), lambda i: (i, 0)),
        out_shape=jax.ShapeDtypeStruct((e, D), jnp.float32),
    )(arr8, consts)


NGRP = BLK // 16


def _sc_body(dst_h, src_h, ew_h, A_h, B_h, T_h,
             osum, osq, omin, omax, ocnt,
             accs, accq, accn, accx, cntv, dstblk, gaccb,
             ev_v, dv_v, svg, ewg, arows, brows, trows,
             pend_dv, pend_e, sew, sem):
    E = dst_h.shape[0]
    N = A_h.shape[0]
    nblk = E // BLK
    cid = lax.axis_index("c")
    sid = lax.axis_index("s")
    wid = sid * 2 + cid

    zero16 = jnp.zeros((16,), jnp.float32)
    inf16 = jnp.full((16,), jnp.inf, jnp.float32)
    e0vec = jnp.where(lax.iota(jnp.int32, 16) == 0,
                      jnp.float32(1.0), jnp.float32(0.0))
    iota16 = lax.iota(jnp.int32, 16)

    def _work(i, lo):
        j = i & 15

        @pl.when(j == 0)
        def _():
            head = pl.multiple_of(i & ~15, 16)
            dvv = jnp.zeros((16,), jnp.int32)
            evv = jnp.zeros((16,), jnp.int32)
            for t in range(16):
                dvv = jnp.where(iota16 == t, pend_dv[head + t], dvv)
                evv = jnp.where(iota16 == t, pend_e[head + t], evv)
            dv_v[...] = jnp.minimum(dvv, N - 1)
            ev_v[...] = evv
            cp_s = pltpu.async_copy(src_h.at[ev_v], svg, sem)
            cp_w = pltpu.async_copy(ew_h.at[ev_v], ewg, sem)
            cp_a = pltpu.async_copy(A_h.at[dv_v], arows, sem)
            cp_t = pltpu.async_copy(T_h.at[ev_v], trows, sem)
            cp_s.wait()
            cp_b = pltpu.async_copy(B_h.at[svg], brows, sem)
            cp_w.wait()
            ewr = ewg[...]
            for t in range(16):
                sew[t] = ewr[t]
            cp_a.wait()
            cp_t.wait()
            cp_b.wait()

        dl = pend_dv[i] - lo
        ew = sew[j]
        plsc.addupdate(cntv.at[dl], e0vec)
        for q in range(D // 16):
            sl = pl.ds(q * 16, 16)
            v = ew * (arows[j, sl] + brows[j, sl]) + trows[j, sl]
            plsc.addupdate(accs.at[dl, sl], v)
            plsc.addupdate(accq.at[dl, sl], v * v)
            accn[dl, sl] = jnp.minimum(accn[dl, sl], v)
            accx[dl, sl] = jnp.maximum(accx[dl, sl], v)

    def _popcount(x):
        x = x - ((x >> 1) & 0x55555555)
        x = (x & 0x33333333) + ((x >> 2) & 0x33333333)
        x = (x + (x >> 4)) & 0x0F0F0F0F
        return (x * 0x01010101) >> 24

    for p in range(2):
        lo = (p * NW + wid) * R
        hi = lo + R

        def _init_acc(r, _):
            for q in range(D // 16):
                sl = pl.ds(q * 16, 16)
                accs[r, sl] = zero16
                accq[r, sl] = zero16
                accn[r, sl] = inf16
                accx[r, sl] = -inf16
            cntv[r, :] = zero16
            return 0
        lax.fori_loop(0, R + 1, _init_acc, 0)

        def _block(b, n):
            boff = pl.multiple_of(b * BLK, BLK)
            pltpu.sync_copy(dst_h.at[pl.ds(boff, BLK)], dstblk)
            gacc = jnp.zeros((16,), jnp.int32)
            for k in range(NGRP):
                dv = dstblk[pl.ds(k * 16, 16)]
                msk = jnp.where((dv >= lo) & (dv < hi),
                                jnp.int32(1 << k), jnp.int32(0))
                gacc = gacc | msk
            gaccb[...] = gacc
            gv = gaccb[...]
            for l in range(16):
                gl0 = gv[l]

                def _bit(_, carry):
                    gl, nn = carry
                    bit = gl & (-gl)
                    fb = lax.bitcast_convert_type(
                        bit.astype(jnp.float32), jnp.int32)
                    k = (fb >> 23) - 127
                    ko = pl.multiple_of(k * 16, 16)
                    dvk = dstblk[pl.ds(ko, 16)]
                    pend_dv[nn] = dvk[l]
                    pend_e[nn] = boff + k * 16 + l
                    return (gl ^ bit, nn + 1)

                (_, n) = lax.fori_loop(0, _popcount(gl0), _bit, (gl0, n))
            nfull16 = n & ~15

            def _w(i, _):
                _work(i, lo)
                return 0
            lax.fori_loop(0, nfull16, _w, 0)
            nrem = n - nfull16

            def _mv(i, _):
                pend_dv[i] = pend_dv[nfull16 + i]
                pend_e[i] = pend_e[nfull16 + i]
                return 0
            lax.fori_loop(0, nrem, _mv, 0)
            return nrem

        n = lax.fori_loop(0, nblk, _block, jnp.int32(0))

        @pl.when(n > 0)
        def _():
            def _pad(i, _):
                @pl.when(i >= n)
                def _():
                    pend_dv[i] = lo + R
                    pend_e[i] = 0
                return 0
            lax.fori_loop(0, 16, _pad, 0)

            def _w2(i, _):
                _work(i, lo)
                return 0
            lax.fori_loop(0, 16, _w2, 0)

        obase = (p * NW + wid) * R
        pltpu.sync_copy(accs.at[pl.ds(0, R)], osum.at[pl.ds(obase, R)])
        pltpu.sync_copy(accq.at[pl.ds(0, R)], osq.at[pl.ds(obase, R)])
        pltpu.sync_copy(accn.at[pl.ds(0, R)], omin.at[pl.ds(obase, R)])
        pltpu.sync_copy(accx.at[pl.ds(0, R)], omax.at[pl.ds(obase, R)])
        pltpu.sync_copy(cntv.at[pl.ds(0, R)], ocnt.at[pl.ds(obase, R)])


def _sc_agg(dst, src, ew, A, B, T):
    ntot = 2 * NW * R
    outs = (
        jax.ShapeDtypeStruct((ntot, D), jnp.float32),
        jax.ShapeDtypeStruct((ntot, D), jnp.float32),
        jax.ShapeDtypeStruct((ntot, D), jnp.float32),
        jax.ShapeDtypeStruct((ntot, D), jnp.float32),
        jax.ShapeDtypeStruct((ntot, 16), jnp.float32),
    )
    mesh = plsc.VectorSubcoreMesh(core_axis_name="c", subcore_axis_name="s",
                                  num_cores=2, num_subcores=16)
    scratch = [
        pltpu.VMEM((R + 8, D), jnp.float32),
        pltpu.VMEM((R + 8, D), jnp.float32),
        pltpu.VMEM((R + 8, D), jnp.float32),
        pltpu.VMEM((R + 8, D), jnp.float32),
        pltpu.VMEM((R + 8, 16), jnp.float32),
        pltpu.VMEM((BLK,), jnp.int32),
        pltpu.VMEM((16,), jnp.int32),
        pltpu.VMEM((16,), jnp.int32),
        pltpu.VMEM((16,), jnp.int32),
        pltpu.VMEM((16,), jnp.int32),
        pltpu.VMEM((16,), jnp.float32),
        pltpu.VMEM((16, D), jnp.float32),
        pltpu.VMEM((16, D), jnp.float32),
        pltpu.VMEM((16, D), jnp.float32),
        pltpu.SMEM((BLK + 16,), jnp.int32),
        pltpu.SMEM((BLK + 16,), jnp.int32),
        pltpu.SMEM((16,), jnp.float32),
        pltpu.SemaphoreType.DMA,
    ]
    fn = pl.kernel(_sc_body, out_type=outs, mesh=mesh, scratch_types=scratch)
    return fn(dst, src, ew, A, B, T)


def _post_body(m_ref, x_ref, s_ref, q_ref, mn_ref, mx_ref, c_ref,
               pm_ref, pagg_ref, pamp_ref, patt_ref, bias_ref,
               wih_ref, whh_ref, o_ref):
    cnt = c_ref[:, 0:1]
    deg = jnp.maximum(cnt, 1.0)
    inv = 1.0 / deg
    mean = s_ref[...] * inv
    msq = q_ref[...] * inv
    var = jnp.maximum(msq - mean * mean, 0.0)
    std = jnp.sqrt(var + 1e-5)
    mn = mn_ref[...]
    mn = jnp.where(jnp.isfinite(mn), mn, 0.0)
    mx = mx_ref[...]
    mx = jnp.where(jnp.isfinite(mx), mx, 0.0)
    agg = jnp.concatenate([mean, mn, mx, std], axis=1)
    ld = jnp.log(deg + 1.0)
    out = jnp.dot(m_ref[...], pm_ref[...], preferred_element_type=jnp.float32)
    out = out + jnp.dot(agg, pagg_ref[...], preferred_element_type=jnp.float32)
    out = out + jnp.dot(agg * (ld / _AVG_LOG), pamp_ref[...],
                        preferred_element_type=jnp.float32)
    out = out + jnp.dot(agg * (_AVG_LOG / ld), patt_ref[...],
                        preferred_element_type=jnp.float32)
    out = out + bias_ref[0:1, :]
    gi = jnp.dot(out, wih_ref[...], preferred_element_type=jnp.float32)
    gh = jnp.dot(x_ref[...], whh_ref[...], preferred_element_type=jnp.float32)
    i_r = gi[:, 0:D] + bias_ref[1:2, :]
    i_z = gi[:, D:2 * D] + bias_ref[2:3, :]
    i_n = gi[:, 2 * D:] + bias_ref[3:4, :]
    h_r = gh[:, 0:D] + bias_ref[4:5, :]
    h_z = gh[:, D:2 * D] + bias_ref[5:6, :]
    h_n = gh[:, 2 * D:] + bias_ref[6:7, :]
    r = jax.nn.sigmoid(i_r + h_r)
    z = jax.nn.sigmoid(i_z + h_z)
    nn_ = jnp.tanh(i_n + r * h_n)
    o_ref[...] = (1.0 - z) * nn_ + z * x_ref[...]


def _post(m, x, s, q, mn, mx, cnt128, w_post, b_post, w_ih, w_hh, b_ih, b_hh):
    n = x.shape[0]
    bn = 1000 if n % 1000 == 0 else n
    grid = n // bn
    pm = w_post[0:D]
    pagg = w_post[D:5 * D]
    pamp = w_post[5 * D:9 * D]
    patt = w_post[9 * D:13 * D]
    bias = jnp.zeros((8, D), jnp.float32)
    bias = bias.at[0].set(b_post)
    bias = bias.at[1:4].set(b_ih.reshape(3, D))
    bias = bias.at[4:7].set(b_hh.reshape(3, D))
    row = pl.BlockSpec((bn, D), lambda i: (i, 0))
    f = lambda shape: pl.BlockSpec(shape, lambda i: (0, 0))
    return pl.pallas_call(
        _post_body,
        grid=(grid,),
        in_specs=[row, row, row, row, row, row, row,
                  f((D, D)), f((4 * D, D)), f((4 * D, D)), f((4 * D, D)),
                  f((8, D)), f((D, 3 * D)), f((D, 3 * D))],
        out_specs=row,
        out_shape=jax.ShapeDtypeStruct((n, D), jnp.float32),
    )(m, x, s, q, mn, mx, cnt128, pm, pagg, pamp, patt, bias,
      w_ih.T, w_hh.T)


@jax.jit
def kernel(x, edge_index, edge_attr, weight, W_edge, b_edge, W_pre, b_pre,
           W_post, b_post, w_ih, w_hh, b_ih, b_hh):
    n = x.shape[0]
    src = edge_index[0]
    dst = edge_index[1]
    wp1 = W_pre[0:D]
    wp2 = W_pre[D:2 * D]
    wp3 = W_pre[2 * D:]
    m, A, B = _pre(x, weight[0], wp1, wp2)
    consts = _consts(W_edge, b_edge, b_pre, wp3)
    e = edge_attr.shape[0]
    ew = edge_attr[:, 3]
    arr8 = jnp.concatenate([
        edge_attr[:, 0:3].T, jnp.ones((1, e), jnp.float32),
        ew[None, :], jnp.zeros((3, e), jnp.float32)], axis=0)
    T = _tmat(arr8, consts)
    s, q, mn, mx, cnt = _sc_agg(dst, src, ew, A, B, T)
    cnt128 = jnp.pad(cnt[:n], ((0, 0), (0, D - 16)))
    return _post(m, x, s[:n], q[:n], mn[:n], mx[:n], cnt128,
                 W_post, b_post, w_ih, w_hh, b_ih, b_hh)
